# baseline (speedup 1.0000x reference)
import functools

import jax
import jax.numpy as jnp
from jax import lax
from jax.experimental import pallas as pl
from jax.experimental.pallas import tpu as pltpu
from jax.experimental.pallas import tpu_sc as plsc

N = 50000
E = 800000
IN_FEATS = 128
H = 64
HH = 32
NC = 2
NS = 16
NT = N // NS
RC = 125
RCD = 625
EC = 128
ECL = 125
BCH = 16
EPT = E // NS
NCHUNK = EPT // ECL
NBLK = NCHUNK // BCH
EPT2 = E // (NC * NS)
E2CHUNK = EPT2 // ECL

THETAS = ((3.0, -3.0, 0.75), (0.0, 3.0, -1.5), (0.0, 0.0, 0.75))

_SC_MESH = plsc.VectorSubcoreMesh(core_axis_name="c", subcore_axis_name="s")
_SC_PARAMS = pltpu.CompilerParams(use_tc_tiling_on_sc=False)


@functools.partial(
    pl.kernel,
    out_type=jax.ShapeDtypeStruct((NC, NS, NT, 16), jnp.float32),
    mesh=_SC_MESH,
    compiler_params=_SC_PARAMS,
    scratch_types=dict(
        deg16=pltpu.VMEM_SHARED((N, 16), jnp.float32),
        ones=pltpu.VMEM((ECL, 16), jnp.float32),
        zbuf=pltpu.VMEM((RCD, 16), jnp.float32),
        didx0=pltpu.VMEM((ECL,), jnp.int32),
        didx1=pltpu.VMEM((ECL,), jnp.int32),
        dsem0=pltpu.SemaphoreType.DMA,
        dsem1=pltpu.SemaphoreType.DMA,
    ),
)
def _deg_kernel(ei_hbm, out_hbm, deg16, ones, zbuf, didx0, didx1,
                dsem0, dsem1):
    c = lax.axis_index("c")
    s = lax.axis_index("s")
    didx = (didx0, didx1)
    dsems = (dsem0, dsem1)

    one16 = jnp.full((16,), 1.0, jnp.float32)
    zero16 = jnp.zeros((16,), jnp.float32)

    def init_row(r, _):
        ones[r, pl.ds(0, 16)] = one16
        return 0

    lax.fori_loop(0, ECL, init_row, 0)

    def zero_row(r, _):
        zbuf[r, pl.ds(0, 16)] = zero16
        return 0

    lax.fori_loop(0, RCD, zero_row, 0)

    def zero_chunk(j, _):
        pltpu.sync_copy(zbuf, deg16.at[pl.ds(s * NT + j * RCD, RCD)])
        return 0

    lax.fori_loop(0, NT // RCD, zero_chunk, 0)

    plsc.subcore_barrier()

    cbase = (c * NS + s) * E2CHUNK

    def idx_load(i, p):
        return pltpu.async_copy(ei_hbm.at[1, cbase + i], didx[p], dsems[p])

    idx_load(0, 0)

    def edge_pair(j, _):
        i = 2 * j
        idx_load(i + 1, 1)
        pltpu.make_async_copy(ei_hbm.at[1, cbase + i], didx0, dsem0).wait()
        pltpu.sync_copy(ones, deg16.at[didx0], add=True)

        @pl.when(j < E2CHUNK // 2 - 1)
        def _():
            idx_load(i + 2, 0)

        pltpu.make_async_copy(ei_hbm.at[1, cbase + i + 1], didx1,
                              dsem1).wait()
        pltpu.sync_copy(ones, deg16.at[didx1], add=True)
        return 0

    lax.fori_loop(0, E2CHUNK // 2, edge_pair, 0)

    plsc.subcore_barrier()

    pltpu.sync_copy(deg16.at[pl.ds(s * NT, NT)], out_hbm.at[c, s])


def _lap_phase0(s, feat_ref, fs_ref, dinvb_hbm, fbuf, fsbuf, dbuf, agg):
    zero16 = jnp.zeros((16,), jnp.float32)

    def zero_row(r, _):
        fsbuf[r, pl.ds(0, 16)] = zero16
        fsbuf[r, pl.ds(16, 16)] = zero16
        return 0

    lax.fori_loop(0, RC, zero_row, 0)

    def zero_chunk(j, _):
        pltpu.sync_copy(fsbuf, agg.at[pl.ds(s * NT + j * RC, RC)])
        return 0

    lax.fori_loop(0, NT // RC, zero_chunk, 0)

    def chunk(j, _):
        base = s * NT + j * RC
        pltpu.sync_copy(feat_ref.at[pl.ds(base, RC)], fbuf)
        pltpu.sync_copy(dinvb_hbm.at[pl.ds(base, RC)], dbuf)

        def row(r, _):
            fsbuf[r, pl.ds(0, 16)] = (
                fbuf[r, pl.ds(0, 16)] * dbuf[r, pl.ds(0, 16)])
            fsbuf[r, pl.ds(16, 16)] = (
                fbuf[r, pl.ds(16, 16)] * dbuf[r, pl.ds(16, 16)])
            return 0

        lax.fori_loop(0, RC, row, 0)
        pltpu.sync_copy(fsbuf, fs_ref.at[pl.ds(base, RC)])
        return 0

    lax.fori_loop(0, NT // RC, chunk, 0)


def _lap_phase1(s, fs_ref, ei_hbm, srcb, dstb, rows, gsems, ssems, isem,
                agg):
    cbase = s * NBLK

    def load_idx_sync(blk, q):
        r = (cbase + blk) * BCH
        pltpu.sync_copy(ei_hbm.at[0, pl.ds(r, BCH)], srcb[q])
        pltpu.sync_copy(ei_hbm.at[1, pl.ds(r, BCH)], dstb[q])

    def load_idx_async(blk, q):
        r = (cbase + blk) * BCH
        d1 = pltpu.async_copy(ei_hbm.at[0, pl.ds(r, BCH)], srcb[q], isem)
        d2 = pltpu.async_copy(ei_hbm.at[1, pl.ds(r, BCH)], dstb[q], isem)
        return (d1, d2)

    def gather(q, i, p):
        return pltpu.async_copy(fs_ref.at[srcb[q].at[i]], rows[p], gsems[p])

    def do_block(q, next_blk, qn, last):
        descs = None
        sdescs = [None, None]
        for i in range(BCH):
            p = i % 2
            if i == 0 and not last:
                descs = load_idx_async(next_blk, qn)
            if sdescs[p ^ 1] is not None:
                sdescs[p ^ 1].wait()
                sdescs[p ^ 1] = None
            if i < BCH - 1:
                gather(q, i + 1, p ^ 1)
            elif not last:
                descs[0].wait()
                descs[1].wait()
                gather(qn, 0, p ^ 1)
            pltpu.make_async_copy(fs_ref.at[srcb[q].at[i]], rows[p],
                                  gsems[p]).wait()
            sdescs[p] = pltpu.async_copy(rows[p], agg.at[dstb[q].at[i]],
                                         ssems[p], add=True)
        for sd in sdescs:
            if sd is not None:
                sd.wait()

    load_idx_sync(0, 0)
    gather(0, 0, 0)

    def dblk(jj, _):
        do_block(0, 2 * jj + 1, 1, False)
        do_block(1, 2 * jj + 2, 0, False)
        return 0

    lax.fori_loop(0, (NBLK - 1) // 2, dblk, 0)
    do_block(0, 0, 0, True)


def _lap_phase2(s, feat_ref, fn_ref, dinvb_hbm, fbuf, fsbuf, abuf, dbuf, agg):
    def chunk(j, _):
        base = s * NT + j * RC
        pltpu.sync_copy(feat_ref.at[pl.ds(base, RC)], fbuf)
        pltpu.sync_copy(dinvb_hbm.at[pl.ds(base, RC)], dbuf)
        pltpu.sync_copy(agg.at[pl.ds(base, RC)], abuf)

        def row(r, _):
            fsbuf[r, pl.ds(0, 16)] = (
                fbuf[r, pl.ds(0, 16)]
                - abuf[r, pl.ds(0, 16)] * dbuf[r, pl.ds(0, 16)])
            fsbuf[r, pl.ds(16, 16)] = (
                fbuf[r, pl.ds(16, 16)]
                - abuf[r, pl.ds(16, 16)] * dbuf[r, pl.ds(16, 16)])
            return 0

        lax.fori_loop(0, RC, row, 0)
        pltpu.sync_copy(fsbuf, fn_ref.at[pl.ds(base, RC)])
        return 0

    lax.fori_loop(0, NT // RC, chunk, 0)


@functools.partial(
    pl.kernel,
    out_type=(
        jax.ShapeDtypeStruct((N, HH), jnp.float32),
        jax.ShapeDtypeStruct((N, HH), jnp.float32),
        jax.ShapeDtypeStruct((N, HH), jnp.float32),
        jax.ShapeDtypeStruct((N, HH), jnp.float32),
    ),
    mesh=_SC_MESH,
    compiler_params=_SC_PARAMS,
    scratch_types=dict(
        agg=pltpu.VMEM_SHARED((N, HH), jnp.float32),
        rows0=pltpu.VMEM((ECL, HH), jnp.float32),
        rows1=pltpu.VMEM((ECL, HH), jnp.float32),
        abuf=pltpu.VMEM((RC, HH), jnp.float32),
        dbuf=pltpu.VMEM((RC, HH), jnp.float32),
        srcb0=pltpu.VMEM((BCH, ECL), jnp.int32),
        srcb1=pltpu.VMEM((BCH, ECL), jnp.int32),
        dstb0=pltpu.VMEM((BCH, ECL), jnp.int32),
        dstb1=pltpu.VMEM((BCH, ECL), jnp.int32),
        gsem0=pltpu.SemaphoreType.DMA,
        gsem1=pltpu.SemaphoreType.DMA,
        ssem0=pltpu.SemaphoreType.DMA,
        ssem1=pltpu.SemaphoreType.DMA,
        isem=pltpu.SemaphoreType.DMA,
    ),
)
def _lap_kernel(flo_hbm, fhi_hbm, dinvb_hbm, ei_hbm,
                fnlo_hbm, fnhi_hbm, fslo_hbm, fshi_hbm,
                agg, rows0, rows1, abuf, dbuf,
                srcb0, srcb1, dstb0, dstb1, gsem0, gsem1, ssem0, ssem1,
                isem):
    c = lax.axis_index("c")
    s = lax.axis_index("s")
    srcb = (srcb0, srcb1)
    dstb = (dstb0, dstb1)
    rows = (rows0, rows1)
    gsems = (gsem0, gsem1)
    ssems = (ssem0, ssem1)

    @pl.when(c == 0)
    def _():
        _lap_phase0(s, flo_hbm, fslo_hbm, dinvb_hbm, rows0, rows1, dbuf, agg)

    @pl.when(c == 1)
    def _():
        _lap_phase0(s, fhi_hbm, fshi_hbm, dinvb_hbm, rows0, rows1, dbuf, agg)

    plsc.subcore_barrier()

    @pl.when(c == 0)
    def _():
        _lap_phase1(s, fslo_hbm, ei_hbm, srcb, dstb, rows, gsems, ssems,
                    isem, agg)

    @pl.when(c == 1)
    def _():
        _lap_phase1(s, fshi_hbm, ei_hbm, srcb, dstb, rows, gsems, ssems,
                    isem, agg)

    plsc.subcore_barrier()

    @pl.when(c == 0)
    def _():
        _lap_phase2(s, flo_hbm, fnlo_hbm, dinvb_hbm, rows0, rows1, abuf,
                    dbuf, agg)

    @pl.when(c == 1)
    def _():
        _lap_phase2(s, fhi_hbm, fnhi_hbm, dinvb_hbm, rows0, rows1, abuf,
                    dbuf, agg)


def _ln(x, g, b):
    m = jnp.mean(x, axis=-1, keepdims=True)
    v = jnp.mean((x - m) ** 2, axis=-1, keepdims=True)
    return (x - m) * jax.lax.rsqrt(v + 1e-5) * g + b


def _mlp_body(x_ref, w1_ref, b1_ref, g1_ref, be1_ref, w2_ref, b2_ref,
              g2_ref, be2_ref, hlo_ref, hhi_ref):
    x = x_ref[...]
    h = jax.lax.dot_general(x, w1_ref[...], (((1,), (1,)), ((), ())),
                            preferred_element_type=jnp.float32)
    h = h + b1_ref[...]
    h = jax.nn.relu(_ln(h, g1_ref[...], be1_ref[...]))
    h = jax.lax.dot_general(h, w2_ref[...], (((1,), (1,)), ((), ())),
                            preferred_element_type=jnp.float32)
    h = h + b2_ref[...]
    h = jax.nn.relu(_ln(h, g2_ref[...], be2_ref[...]))
    hlo_ref[...] = h[:, :HH]
    hhi_ref[...] = h[:, HH:]


_BLK = 2000
_GRID = N // _BLK
_BLKH = 2000
_GRIDH = N // _BLKH


def _mlp_call(in_feat, W1, b1, g1, be1, W2, b2, g2, be2):
    full = lambda shp: pl.BlockSpec(shp, lambda i: (0, 0))
    return pl.pallas_call(
        _mlp_body,
        grid=(_GRID,),
        in_specs=[
            pl.BlockSpec((_BLK, IN_FEATS), lambda i: (i, 0)),
            full((H, IN_FEATS)), full((1, H)), full((1, H)), full((1, H)),
            full((H, H)), full((1, H)), full((1, H)), full((1, H)),
        ],
        out_specs=[
            pl.BlockSpec((_BLK, HH), lambda i: (i, 0)),
            pl.BlockSpec((_BLK, HH), lambda i: (i, 0)),
        ],
        out_shape=[
            jax.ShapeDtypeStruct((N, HH), jnp.float32),
            jax.ShapeDtypeStruct((N, HH), jnp.float32),
        ],
    )(in_feat, W1, b1[None, :], g1[None, :], be1[None, :],
      W2, b2[None, :], g2[None, :], be2[None, :])


def _head_body(hlo_ref, hhi_ref, f1lo_ref, f1hi_ref, f2lo_ref, f2hi_ref,
               th_ref, wa1b_ref, ba1r_ref, g2_ref, ba2_ref, s3_ref,
               w3_ref, b3_ref, w4_ref, b4_ref, out_ref):
    x = jnp.concatenate(
        [hlo_ref[...], hhi_ref[...], f1lo_ref[...], f1hi_ref[...],
         f2lo_ref[...], f2hi_ref[...]], axis=1)

    def mm(a, b):
        return jax.lax.dot_general(a, b, (((1,), (0,)), ((), ())),
                                   preferred_element_type=jnp.float32)

    hs = mm(x, th_ref[...])
    a = jnp.tanh(mm(hs, wa1b_ref[...]) + ba1r_ref[...])
    d3 = jnp.tanh(mm(a, g2_ref[...]) + ba2_ref[...])
    he = hs * (1.0 + mm(d3, s3_ref[...]))
    hr = jax.nn.relu(
        jax.lax.dot_general(he, w3_ref[...], (((1,), (1,)), ((), ())),
                            preferred_element_type=jnp.float32) + b3_ref[...])
    o = jax.lax.dot_general(
        hr, w4_ref[...], (((1,), (1,)), ((), ())),
        preferred_element_type=jnp.float32) + b4_ref[...]
    out_ref[...] = o[:, :2]


def _head_call(hlo, hhi, f1lo, f1hi, f2lo, f2hi, Wa1, ba1, Wa2, ba2,
               W3, b3, W4p, b4p):
    eye3 = jnp.eye(3, dtype=jnp.float32)
    th_mat = jnp.array(THETAS, jnp.float32).T
    TH = jnp.kron(th_mat, jnp.eye(H, dtype=jnp.float32))
    WA1B = jnp.kron(eye3, Wa1.T)
    ba1r = jnp.tile(ba1, 3)[None, :]
    G2 = jnp.kron(eye3, Wa2[0][:, None])
    S3 = jnp.kron(eye3, jnp.ones((1, H), jnp.float32))
    full = lambda shp: pl.BlockSpec(shp, lambda i: (0, 0))
    blk = lambda: pl.BlockSpec((_BLKH, HH), lambda i: (i, 0))
    return pl.pallas_call(
        _head_body,
        grid=(_GRIDH,),
        in_specs=[
            blk(), blk(), blk(), blk(), blk(), blk(),
            full((3 * H, 3 * H)), full((3 * H, 96)), full((1, 96)),
            full((96, 3)), full((1, 1)), full((3, 3 * H)),
            full((H, 3 * H)), full((1, H)),
            full((128, H)), full((1, 128)),
        ],
        out_specs=pl.BlockSpec((_BLKH, 2), lambda i: (i, 0)),
        out_shape=jax.ShapeDtypeStruct((N, 2), jnp.float32),
    )(hlo, hhi, f1lo, f1hi, f2lo, f2hi,
      TH, WA1B, ba1r, G2, ba2[None, :].reshape(1, 1), S3,
      W3, b3[None, :], W4p, b4p[None, :])


def kernel(in_feat, edge_index, W1, b1, g1, be1, W2, b2, g2, be2,
           Wa1, ba1, Wa2, ba2, W3, b3, W4, b4):
    ei3 = edge_index.reshape(2, E // ECL, ECL)

    deg_part = _deg_kernel(ei3)
    deg = deg_part[..., 0].reshape(NC, N).sum(axis=0)
    d_invsqrt = jnp.clip(deg, 1.0, None) ** -0.5
    dinvb = jnp.broadcast_to(d_invsqrt[:, None], (N, HH))

    hlo, hhi = _mlp_call(in_feat, W1, b1, g1, be1, W2, b2, g2, be2)

    f1lo, f1hi, _, _ = _lap_kernel(hlo, hhi, dinvb, ei3)
    f2lo, f2hi, _, _ = _lap_kernel(f1lo, f1hi, dinvb, ei3)

    W4p = jnp.zeros((128, H), jnp.float32).at[:2].set(W4)
    b4p = jnp.zeros((128,), jnp.float32).at[:2].set(b4)
    return _head_call(hlo, hhi, f1lo, f1hi, f2lo, f2hi,
                      Wa1, ba1, Wa2, ba2, W3, b3, W4p, b4p)


# speedup vs baseline: 1.0395x; 1.0395x over previous
import functools

import jax
import jax.numpy as jnp
from jax import lax
from jax.experimental import pallas as pl
from jax.experimental.pallas import tpu as pltpu
from jax.experimental.pallas import tpu_sc as plsc

N = 50000
E = 800000
IN_FEATS = 128
H = 64
HH = 32
NC = 2
NS = 16
NT = N // NS
RC = 125
RCD = 625
EC = 128
ECL = 125
BCH = 16
EPT = E // NS
NCHUNK = EPT // ECL
NBLK = NCHUNK // BCH
EPT2 = E // (NC * NS)
E2CHUNK = EPT2 // ECL

THETAS = ((3.0, -3.0, 0.75), (0.0, 3.0, -1.5), (0.0, 0.0, 0.75))

_SC_MESH = plsc.VectorSubcoreMesh(core_axis_name="c", subcore_axis_name="s")
_SC_PARAMS = pltpu.CompilerParams(use_tc_tiling_on_sc=False,
                                  needs_layout_passes=False)


@functools.partial(
    pl.kernel,
    out_type=jax.ShapeDtypeStruct((NC, NS, NT), jnp.float32),
    mesh=_SC_MESH,
    compiler_params=_SC_PARAMS,
    scratch_types=dict(
        deg16=pltpu.VMEM_SHARED((N, 16), jnp.float32),
        ones=pltpu.VMEM((ECL, 16), jnp.float32),
        zbuf=pltpu.VMEM((RCD, 16), jnp.float32),
        didx0=pltpu.VMEM((ECL,), jnp.int32),
        didx1=pltpu.VMEM((ECL,), jnp.int32),
        dbig=pltpu.VMEM((NT, 16), jnp.float32),
        outb=pltpu.VMEM((3136,), jnp.float32),
        dsem0=pltpu.SemaphoreType.DMA,
        dsem1=pltpu.SemaphoreType.DMA,
    ),
)
def _deg_kernel(ei_hbm, out_hbm, deg16, ones, zbuf, didx0, didx1,
                dbig, outb, dsem0, dsem1):
    c = lax.axis_index("c")
    s = lax.axis_index("s")
    didx = (didx0, didx1)
    dsems = (dsem0, dsem1)

    one16 = jnp.full((16,), 1.0, jnp.float32)
    zero16 = jnp.zeros((16,), jnp.float32)

    def init_row(r, _):
        ones[r, pl.ds(0, 16)] = one16
        return 0

    lax.fori_loop(0, ECL, init_row, 0)

    def zero_row(r, _):
        zbuf[r, pl.ds(0, 16)] = zero16
        return 0

    lax.fori_loop(0, RCD, zero_row, 0)

    def zero_chunk(j, _):
        pltpu.sync_copy(zbuf, deg16.at[pl.ds(s * NT + j * RCD, RCD)])
        return 0

    lax.fori_loop(0, NT // RCD, zero_chunk, 0)

    plsc.subcore_barrier()

    cbase = (c * NS + s) * E2CHUNK

    def idx_load(i, p):
        return pltpu.async_copy(ei_hbm.at[1, cbase + i], didx[p], dsems[p])

    idx_load(0, 0)

    def edge_pair(j, _):
        i = 2 * j
        idx_load(i + 1, 1)
        pltpu.make_async_copy(ei_hbm.at[1, cbase + i], didx0, dsem0).wait()
        pltpu.sync_copy(ones, deg16.at[didx0], add=True)

        @pl.when(j < E2CHUNK // 2 - 1)
        def _():
            idx_load(i + 2, 0)

        pltpu.make_async_copy(ei_hbm.at[1, cbase + i + 1], didx1,
                              dsem1).wait()
        pltpu.sync_copy(ones, deg16.at[didx1], add=True)
        return 0

    lax.fori_loop(0, E2CHUNK // 2, edge_pair, 0)

    plsc.subcore_barrier()

    pltpu.sync_copy(deg16.at[pl.ds(s * NT, NT)], dbig)
    lane = lax.iota(jnp.int32, 16)
    zero16i = jnp.zeros((16,), jnp.int32)

    def extract(g, _):
        ridx = jnp.minimum(g * 16 + lane, NT - 1)
        outb[pl.ds(g * 16, 16)] = plsc.load_gather(dbig, [ridx, zero16i])
        return 0

    lax.fori_loop(0, 3136 // 16, extract, 0)
    pltpu.sync_copy(outb.at[pl.ds(0, NT)], out_hbm.at[c, s])


def _lap_phase0(s, feat_ref, fs_ref, dinvb_hbm, fbuf, fsbuf, dbuf, agg):
    zero16 = jnp.zeros((16,), jnp.float32)

    def zero_row(r, _):
        fsbuf[r, pl.ds(0, 16)] = zero16
        fsbuf[r, pl.ds(16, 16)] = zero16
        return 0

    lax.fori_loop(0, RC, zero_row, 0)

    def zero_chunk(j, _):
        pltpu.sync_copy(fsbuf, agg.at[pl.ds(s * NT + j * RC, RC)])
        return 0

    lax.fori_loop(0, NT // RC, zero_chunk, 0)

    def chunk(j, _):
        base = s * NT + j * RC
        pltpu.sync_copy(feat_ref.at[pl.ds(base, RC)], fbuf)
        pltpu.sync_copy(dinvb_hbm.at[pl.ds(base, RC)], dbuf)

        def row(r, _):
            fsbuf[r, pl.ds(0, 16)] = (
                fbuf[r, pl.ds(0, 16)] * dbuf[r, pl.ds(0, 16)])
            fsbuf[r, pl.ds(16, 16)] = (
                fbuf[r, pl.ds(16, 16)] * dbuf[r, pl.ds(16, 16)])
            return 0

        lax.fori_loop(0, RC, row, 0)
        pltpu.sync_copy(fsbuf, fs_ref.at[pl.ds(base, RC)])
        return 0

    lax.fori_loop(0, NT // RC, chunk, 0)


def _lap_phase1(s, fs_ref, ei_hbm, srcb, dstb, rows, gsems, ssems, isem,
                agg):
    cbase = s * NBLK

    def load_idx_sync(blk, q):
        r = (cbase + blk) * BCH
        pltpu.sync_copy(ei_hbm.at[0, pl.ds(r, BCH)], srcb[q])
        pltpu.sync_copy(ei_hbm.at[1, pl.ds(r, BCH)], dstb[q])

    def load_idx_async(blk, q):
        r = (cbase + blk) * BCH
        d1 = pltpu.async_copy(ei_hbm.at[0, pl.ds(r, BCH)], srcb[q], isem)
        d2 = pltpu.async_copy(ei_hbm.at[1, pl.ds(r, BCH)], dstb[q], isem)
        return (d1, d2)

    def gather(q, i, p):
        return pltpu.async_copy(fs_ref.at[srcb[q].at[i]], rows[p], gsems[p])

    def do_block(q, next_blk, qn, last):
        descs = None
        sdescs = [None, None]
        for i in range(BCH):
            p = i % 2
            if i == 0 and not last:
                descs = load_idx_async(next_blk, qn)
            if sdescs[p ^ 1] is not None:
                sdescs[p ^ 1].wait()
                sdescs[p ^ 1] = None
            if i < BCH - 1:
                gather(q, i + 1, p ^ 1)
            elif not last:
                descs[0].wait()
                descs[1].wait()
                gather(qn, 0, p ^ 1)
            pltpu.make_async_copy(fs_ref.at[srcb[q].at[i]], rows[p],
                                  gsems[p]).wait()
            sdescs[p] = pltpu.async_copy(rows[p], agg.at[dstb[q].at[i]],
                                         ssems[p], add=True)
        for sd in sdescs:
            if sd is not None:
                sd.wait()

    load_idx_sync(0, 0)
    gather(0, 0, 0)

    def dblk(jj, _):
        do_block(0, 2 * jj + 1, 1, False)
        do_block(1, 2 * jj + 2, 0, False)
        return 0

    lax.fori_loop(0, (NBLK - 1) // 2, dblk, 0)
    do_block(0, 0, 0, True)


def _lap_phase2(s, feat_ref, fn_ref, dinvb_hbm, fbuf, fsbuf, abuf, dbuf, agg):
    def chunk(j, _):
        base = s * NT + j * RC
        pltpu.sync_copy(feat_ref.at[pl.ds(base, RC)], fbuf)
        pltpu.sync_copy(dinvb_hbm.at[pl.ds(base, RC)], dbuf)
        pltpu.sync_copy(agg.at[pl.ds(base, RC)], abuf)

        def row(r, _):
            fsbuf[r, pl.ds(0, 16)] = (
                fbuf[r, pl.ds(0, 16)]
                - abuf[r, pl.ds(0, 16)] * dbuf[r, pl.ds(0, 16)])
            fsbuf[r, pl.ds(16, 16)] = (
                fbuf[r, pl.ds(16, 16)]
                - abuf[r, pl.ds(16, 16)] * dbuf[r, pl.ds(16, 16)])
            return 0

        lax.fori_loop(0, RC, row, 0)
        pltpu.sync_copy(fsbuf, fn_ref.at[pl.ds(base, RC)])
        return 0

    lax.fori_loop(0, NT // RC, chunk, 0)


@functools.partial(
    pl.kernel,
    out_type=(
        jax.ShapeDtypeStruct((N, HH), jnp.float32),
        jax.ShapeDtypeStruct((N, HH), jnp.float32),
        jax.ShapeDtypeStruct((N, HH), jnp.float32),
        jax.ShapeDtypeStruct((N, HH), jnp.float32),
        jax.ShapeDtypeStruct((N, HH), jnp.float32),
        jax.ShapeDtypeStruct((N, HH), jnp.float32),
    ),
    mesh=_SC_MESH,
    compiler_params=_SC_PARAMS,
    scratch_types=dict(
        agg=pltpu.VMEM_SHARED((N, HH), jnp.float32),
        rows0=pltpu.VMEM((ECL, HH), jnp.float32),
        rows1=pltpu.VMEM((ECL, HH), jnp.float32),
        abuf=pltpu.VMEM((RC, HH), jnp.float32),
        dbuf=pltpu.VMEM((RC, HH), jnp.float32),
        srcb0=pltpu.VMEM((BCH, ECL), jnp.int32),
        srcb1=pltpu.VMEM((BCH, ECL), jnp.int32),
        dstb0=pltpu.VMEM((BCH, ECL), jnp.int32),
        dstb1=pltpu.VMEM((BCH, ECL), jnp.int32),
        gsem0=pltpu.SemaphoreType.DMA,
        gsem1=pltpu.SemaphoreType.DMA,
        ssem0=pltpu.SemaphoreType.DMA,
        ssem1=pltpu.SemaphoreType.DMA,
        isem=pltpu.SemaphoreType.DMA,
    ),
)
def _lap_kernel(flo_hbm, fhi_hbm, dinvb_hbm, ei_hbm,
                f1lo_hbm, f1hi_hbm, f2lo_hbm, f2hi_hbm, fslo_hbm, fshi_hbm,
                agg, rows0, rows1, abuf, dbuf,
                srcb0, srcb1, dstb0, dstb1, gsem0, gsem1, ssem0, ssem1,
                isem):
    c = lax.axis_index("c")
    s = lax.axis_index("s")
    srcb = (srcb0, srcb1)
    dstb = (dstb0, dstb1)
    rows = (rows0, rows1)
    gsems = (gsem0, gsem1)
    ssems = (ssem0, ssem1)

    def step(feat_lo, feat_hi, fn_lo, fn_hi):
        @pl.when(c == 0)
        def _():
            _lap_phase0(s, feat_lo, fslo_hbm, dinvb_hbm, rows0, rows1,
                        dbuf, agg)

        @pl.when(c == 1)
        def _():
            _lap_phase0(s, feat_hi, fshi_hbm, dinvb_hbm, rows0, rows1,
                        dbuf, agg)

        plsc.subcore_barrier()

        @pl.when(c == 0)
        def _():
            _lap_phase1(s, fslo_hbm, ei_hbm, srcb, dstb, rows, gsems,
                        ssems, isem, agg)

        @pl.when(c == 1)
        def _():
            _lap_phase1(s, fshi_hbm, ei_hbm, srcb, dstb, rows, gsems,
                        ssems, isem, agg)

        plsc.subcore_barrier()

        @pl.when(c == 0)
        def _():
            _lap_phase2(s, feat_lo, fn_lo, dinvb_hbm, rows0, rows1, abuf,
                        dbuf, agg)

        @pl.when(c == 1)
        def _():
            _lap_phase2(s, feat_hi, fn_hi, dinvb_hbm, rows0, rows1, abuf,
                        dbuf, agg)

    step(flo_hbm, fhi_hbm, f1lo_hbm, f1hi_hbm)
    step(f1lo_hbm, f1hi_hbm, f2lo_hbm, f2hi_hbm)


def _ln(x, g, b):
    m = jnp.mean(x, axis=-1, keepdims=True)
    v = jnp.mean((x - m) ** 2, axis=-1, keepdims=True)
    return (x - m) * jax.lax.rsqrt(v + 1e-5) * g + b


def _mlp_body(x_ref, w1_ref, b1_ref, g1_ref, be1_ref, w2_ref, b2_ref,
              g2_ref, be2_ref, hlo_ref, hhi_ref):
    x = x_ref[...]
    h = jax.lax.dot_general(x, w1_ref[...], (((1,), (1,)), ((), ())),
                            preferred_element_type=jnp.float32)
    h = h + b1_ref[...]
    h = jax.nn.relu(_ln(h, g1_ref[...], be1_ref[...]))
    h = jax.lax.dot_general(h, w2_ref[...], (((1,), (1,)), ((), ())),
                            preferred_element_type=jnp.float32)
    h = h + b2_ref[...]
    h = jax.nn.relu(_ln(h, g2_ref[...], be2_ref[...]))
    hlo_ref[...] = h[:, :HH]
    hhi_ref[...] = h[:, HH:]


_BLK = 2000
_GRID = N // _BLK
_BLKH = 2000
_GRIDH = N // _BLKH


def _mlp_call(in_feat, W1, b1, g1, be1, W2, b2, g2, be2):
    full = lambda shp: pl.BlockSpec(shp, lambda i: (0, 0))
    return pl.pallas_call(
        _mlp_body,
        grid=(_GRID,),
        in_specs=[
            pl.BlockSpec((_BLK, IN_FEATS), lambda i: (i, 0)),
            full((H, IN_FEATS)), full((1, H)), full((1, H)), full((1, H)),
            full((H, H)), full((1, H)), full((1, H)), full((1, H)),
        ],
        out_specs=[
            pl.BlockSpec((_BLK, HH), lambda i: (i, 0)),
            pl.BlockSpec((_BLK, HH), lambda i: (i, 0)),
        ],
        out_shape=[
            jax.ShapeDtypeStruct((N, HH), jnp.float32),
            jax.ShapeDtypeStruct((N, HH), jnp.float32),
        ],
    )(in_feat, W1, b1[None, :], g1[None, :], be1[None, :],
      W2, b2[None, :], g2[None, :], be2[None, :])


def _head_body(hlo_ref, hhi_ref, f1lo_ref, f1hi_ref, f2lo_ref, f2hi_ref,
               th_ref, wa1b_ref, ba1r_ref, g2_ref, ba2_ref, s3_ref,
               w3_ref, b3_ref, w4_ref, b4_ref, out_ref):
    x = jnp.concatenate(
        [hlo_ref[...], hhi_ref[...], f1lo_ref[...], f1hi_ref[...],
         f2lo_ref[...], f2hi_ref[...]], axis=1)

    def mm(a, b):
        return jax.lax.dot_general(a, b, (((1,), (0,)), ((), ())),
                                   preferred_element_type=jnp.float32)

    hs = mm(x, th_ref[...])
    a = jnp.tanh(mm(hs, wa1b_ref[...]) + ba1r_ref[...])
    d3 = jnp.tanh(mm(a, g2_ref[...]) + ba2_ref[...])
    he = hs * (1.0 + mm(d3, s3_ref[...]))
    hr = jax.nn.relu(
        jax.lax.dot_general(he, w3_ref[...], (((1,), (1,)), ((), ())),
                            preferred_element_type=jnp.float32) + b3_ref[...])
    o = jax.lax.dot_general(
        hr, w4_ref[...], (((1,), (1,)), ((), ())),
        preferred_element_type=jnp.float32) + b4_ref[...]
    out_ref[...] = o[:, :2]


def _head_call(hlo, hhi, f1lo, f1hi, f2lo, f2hi, Wa1, ba1, Wa2, ba2,
               W3, b3, W4p, b4p):
    eye3 = jnp.eye(3, dtype=jnp.float32)
    th_mat = jnp.array(THETAS, jnp.float32).T
    TH = jnp.kron(th_mat, jnp.eye(H, dtype=jnp.float32))
    WA1B = jnp.kron(eye3, Wa1.T)
    ba1r = jnp.tile(ba1, 3)[None, :]
    G2 = jnp.kron(eye3, Wa2[0][:, None])
    S3 = jnp.kron(eye3, jnp.ones((1, H), jnp.float32))
    full = lambda shp: pl.BlockSpec(shp, lambda i: (0, 0))
    blk = lambda: pl.BlockSpec((_BLKH, HH), lambda i: (i, 0))
    return pl.pallas_call(
        _head_body,
        grid=(_GRIDH,),
        in_specs=[
            blk(), blk(), blk(), blk(), blk(), blk(),
            full((3 * H, 3 * H)), full((3 * H, 96)), full((1, 96)),
            full((96, 3)), full((1, 1)), full((3, 3 * H)),
            full((H, 3 * H)), full((1, H)),
            full((128, H)), full((1, 128)),
        ],
        out_specs=pl.BlockSpec((_BLKH, 2), lambda i: (i, 0)),
        out_shape=jax.ShapeDtypeStruct((N, 2), jnp.float32),
    )(hlo, hhi, f1lo, f1hi, f2lo, f2hi,
      TH, WA1B, ba1r, G2, ba2[None, :].reshape(1, 1), S3,
      W3, b3[None, :], W4p, b4p[None, :])


def kernel(in_feat, edge_index, W1, b1, g1, be1, W2, b2, g2, be2,
           Wa1, ba1, Wa2, ba2, W3, b3, W4, b4):
    ei3 = edge_index.reshape(2, E // ECL, ECL)

    deg_part = _deg_kernel(ei3)
    deg = deg_part.reshape(NC, N).sum(axis=0)
    d_invsqrt = jnp.clip(deg, 1.0, None) ** -0.5
    dinvb = jnp.broadcast_to(d_invsqrt[:, None], (N, HH))

    hlo, hhi = _mlp_call(in_feat, W1, b1, g1, be1, W2, b2, g2, be2)

    f1lo, f1hi, f2lo, f2hi, _, _ = _lap_kernel(hlo, hhi, dinvb, ei3)

    W4p = jnp.zeros((128, H), jnp.float32).at[:2].set(W4)
    b4p = jnp.zeros((128,), jnp.float32).at[:2].set(b4)
    return _head_call(hlo, hhi, f1lo, f1hi, f2lo, f2hi,
                      Wa1, ba1, Wa2, ba2, W3, b3, W4p, b4p)
